# baseline (speedup 1.0000x reference)
import jax
import jax.numpy as jnp
from jax import lax
from jax.experimental import pallas as pl
from jax.experimental.pallas import tpu as pltpu
from jax.experimental.pallas import tpu_sc as plsc

N = 10000
NP = 10240
E = 320000
D = 128
HD = D // 2
G = 64
EPT = E // 16
CH = 125
NCH = EPT // CH
RPT = NP // 16
BLK = 1280
GRID = NP // BLK
NW_DEG = 32

_mesh = plsc.VectorSubcoreMesh(core_axis_name="c", subcore_axis_name="s")


def _deg_body(dst_hbm, degp_hbm, dstb, ones, zbuf, dacc, dsem):
    c = lax.axis_index("c")
    s = lax.axis_index("s")
    wid = c * 16 + s

    for k in range(8):
        ones[pl.ds(16 * k, 16)] = jnp.ones((16,), jnp.float32)

    def _zb(r, carry):
        zbuf[pl.ds(r * 16, 16)] = jnp.zeros((16,), jnp.float32)
        return carry

    lax.fori_loop(0, RPT // 16, _zb, 0)
    pltpu.sync_copy(zbuf, dacc.at[pl.ds(s * RPT, RPT)])
    pltpu.sync_copy(dst_hbm.at[wid], dstb)
    plsc.subcore_barrier()

    def _scat(i, carry):
        for b in range(8):
            pltpu.async_copy(ones.at[pl.ds(0, CH)],
                             dacc.at[dstb.at[i * 8 + b]], dsem, add=True)
        for b in range(8):
            pltpu.make_async_copy(ones.at[pl.ds(0, CH)],
                                  dacc.at[dstb.at[i * 8 + b]], dsem).wait()
        return carry

    lax.fori_loop(0, (EPT // 2) // CH // 8, _scat, 0)
    plsc.subcore_barrier()
    pltpu.sync_copy(dacc.at[pl.ds(s * RPT, RPT)],
                    degp_hbm.at[c, 0, pl.ds(s * RPT, RPT)])


_degree = pl.kernel(
    _deg_body,
    out_type=jax.ShapeDtypeStruct((2, 1, NP), jnp.float32),
    mesh=_mesh,
    scratch_types=[
        pltpu.VMEM((EPT // 2 // CH, CH), jnp.int32),
        pltpu.VMEM((128,), jnp.float32),
        pltpu.VMEM((RPT,), jnp.float32),
        pltpu.VMEM_SHARED((NP,), jnp.float32),
        pltpu.SemaphoreType.DMA,
    ],
)


MBLK = 2000


def _mms_body(x_ref, w_ref, d0_ref, d1_ref, o_ref):
    xs = x_ref[...] * lax.rsqrt(d0_ref[...] + d1_ref[...] + 1.0)
    g = jnp.dot(xs, w_ref[...], preferred_element_type=jnp.float32)
    o_ref[0] = g[:, :HD]
    o_ref[1] = g[:, HD:]


_mm_scale = pl.pallas_call(
    _mms_body,
    grid=(N // MBLK,),
    in_specs=[
        pl.BlockSpec((MBLK, D), lambda i: (i, 0)),
        pl.BlockSpec((D, D), lambda i: (0, 0)),
        pl.BlockSpec((MBLK, 1), lambda i: (i, 0)),
        pl.BlockSpec((MBLK, 1), lambda i: (i, 0)),
    ],
    out_specs=pl.BlockSpec((2, MBLK, HD), lambda i: (0, i, 0)),
    out_shape=jax.ShapeDtypeStruct((2, NP, HD), jnp.float32),
)


def _msg_body(src_hbm, dst_hbm, g_hbm, acc_hbm, srcb, dstb,
              r0, r1, r2, r3, r4, zbuf, acc,
              g0, g1, g2, g3, g4, s0, s1, s2, s3, s4):
    c = lax.axis_index("c")
    s = lax.axis_index("s")
    rows = [r0, r1, r2, r3, r4]
    gsem = [g0, g1, g2, g3, g4]
    ssem = [s0, s1, s2, s3, s4]
    gh = g_hbm.at[c]

    def _zr(r, carry):
        for k in range(HD // 16):
            zbuf[r, pl.ds(16 * k, 16)] = jnp.zeros((16,), jnp.float32)
        return carry

    lax.fori_loop(0, 128, _zr, 0)
    for k in range(RPT // 128):
        pltpu.sync_copy(zbuf, acc.at[pl.ds(s * RPT + k * 128, 128)])
    pltpu.sync_copy(src_hbm.at[s], srcb)
    pltpu.sync_copy(dst_hbm.at[s], dstb)

    def _gather(j, b):
        pltpu.async_copy(gh.at[srcb.at[j]], rows[b], gsem[b])

    def _gather_wait(j, b):
        pltpu.make_async_copy(gh.at[srcb.at[j]], rows[b], gsem[b]).wait()

    def _scatter(j, b):
        pltpu.async_copy(rows[b], acc.at[dstb.at[j]], ssem[b], add=True)

    def _scatter_wait(j, b):
        pltpu.make_async_copy(rows[b], acc.at[dstb.at[j]], ssem[b]).wait()

    _gather(0, 0)
    _gather(1, 1)
    plsc.subcore_barrier()

    def _step(i, carry):
        j = i * 5
        for b in range(5):
            _gather_wait(j + b, b)
            _scatter(j + b, b)
            nb = (b + 2) % 5
            if b < 3:
                @pl.when(i > 0)
                def _(b=b, nb=nb):
                    _scatter_wait(j + b - 3, nb)

                _gather(j + b + 2, nb)
            else:
                _scatter_wait(j + b - 3, nb)

                @pl.when(i < NCH // 5 - 1)
                def _(b=b, nb=nb):
                    _gather(j + b + 2, nb)

        return carry

    lax.fori_loop(0, NCH // 5, _step, 0)
    _scatter_wait(NCH - 3, 2)
    _scatter_wait(NCH - 2, 3)
    _scatter_wait(NCH - 1, 4)
    plsc.subcore_barrier()
    pltpu.sync_copy(acc.at[pl.ds(s * RPT, RPT)],
                    acc_hbm.at[c, pl.ds(s * RPT, RPT)])


_message = pl.kernel(
    _msg_body,
    out_type=jax.ShapeDtypeStruct((2, NP, HD), jnp.float32),
    mesh=_mesh,
    scratch_types=[
        pltpu.VMEM((NCH, CH), jnp.int32),
        pltpu.VMEM((NCH, CH), jnp.int32),
        pltpu.VMEM((CH, HD), jnp.float32),
        pltpu.VMEM((CH, HD), jnp.float32),
        pltpu.VMEM((CH, HD), jnp.float32),
        pltpu.VMEM((CH, HD), jnp.float32),
        pltpu.VMEM((CH, HD), jnp.float32),
        pltpu.VMEM((128, HD), jnp.float32),
        pltpu.VMEM_SHARED((NP, HD), jnp.float32),
        pltpu.SemaphoreType.DMA,
        pltpu.SemaphoreType.DMA,
        pltpu.SemaphoreType.DMA,
        pltpu.SemaphoreType.DMA,
        pltpu.SemaphoreType.DMA,
        pltpu.SemaphoreType.DMA,
        pltpu.SemaphoreType.DMA,
        pltpu.SemaphoreType.DMA,
        pltpu.SemaphoreType.DMA,
        pltpu.SemaphoreType.DMA,
    ],
    compiler_params=pltpu.CompilerParams(use_tc_tiling_on_sc=False),
)


def _fin_body(a_ref, g_ref, d0_ref, d1_ref, bt_ref, b_ref, o_ref):
    i = pl.program_id(0)
    dinv = lax.rsqrt(d0_ref[...] + d1_ref[...] + 1.0)
    sfull = jnp.concatenate([a_ref[0] + g_ref[0], a_ref[1] + g_ref[1]],
                            axis=1)
    out = jnp.maximum(dinv * sfull + b_ref[...], 0.0)
    out = jnp.where(bt_ref[...] < G, out, 0.0)
    gids = lax.broadcasted_iota(jnp.int32, (1, G), 1)
    mask = (bt_ref[...] == gids).astype(jnp.float32)
    part = lax.dot_general(mask, out, (((0,), (0,)), ((), ())),
                           preferred_element_type=jnp.float32)

    @pl.when(i == 0)
    def _():
        o_ref[...] = part

    @pl.when(i != 0)
    def _():
        o_ref[...] = o_ref[...] + part


_finalize = pl.pallas_call(
    _fin_body,
    grid=(GRID,),
    in_specs=[
        pl.BlockSpec((2, BLK, HD), lambda i: (0, i, 0)),
        pl.BlockSpec((2, BLK, HD), lambda i: (0, i, 0)),
        pl.BlockSpec((BLK, 1), lambda i: (i, 0)),
        pl.BlockSpec((BLK, 1), lambda i: (i, 0)),
        pl.BlockSpec((BLK, 1), lambda i: (i, 0)),
        pl.BlockSpec((1, D), lambda i: (0, 0)),
    ],
    out_specs=pl.BlockSpec((G, D), lambda i: (0, 0)),
    out_shape=jax.ShapeDtypeStruct((G, D), jnp.float32),
)


def kernel(x, edge_index, batch_indeces, W, b):
    src2 = edge_index[0].astype(jnp.int32).reshape(16, NCH, CH)
    dst2 = edge_index[1].astype(jnp.int32).reshape(16, NCH, CH)
    dst2h = edge_index[1].astype(jnp.int32).reshape(NW_DEG, NCH // 2, CH)
    bt = jnp.pad(batch_indeces.astype(jnp.int32), (0, NP - N),
                 constant_values=G).reshape(NP, 1)

    degp = _degree(dst2h)
    d0 = degp[0, 0].reshape(NP, 1)
    d1 = degp[1, 0].reshape(NP, 1)
    g = _mm_scale(x, W, d0[:N], d1[:N])
    acc = _message(src2, dst2, g)
    pooled = _finalize(acc, g, d0, d1, bt, b.reshape(1, D))
    return pooled
